# baseline (device time: 71260 ns/iter reference)
import jax
import jax.numpy as jnp
from jax import lax
from jax.experimental import pallas as pl
from jax.experimental.pallas import tpu as pltpu

N_DEV = 16
BP = 128
B = N_DEV * BP
D = 128
HP = 256


def kernel(x, Win0, Wout0, Win1, Wout1, Win2, Wout2):
    def body(x_ref, win0, wout0, win1, wout1, win2, wout2, out_ref,
             xfull, p_ref, rs_buf, ag_send, ag_recv, rs_send, rs_recv):
        my = lax.axis_index("i")
        my_rows = pl.ds(my * BP, BP)

        def all_gather(bi):
            sends = []
            for off in range(1, N_DEV):
                dst = lax.rem(my + off, N_DEV)
                r = pltpu.make_async_remote_copy(
                    src_ref=xfull.at[bi, my_rows, :],
                    dst_ref=xfull.at[bi, my_rows, :],
                    send_sem=ag_send.at[bi, off],
                    recv_sem=ag_recv.at[bi, off],
                    device_id=(dst,),
                    device_id_type=pl.DeviceIdType.MESH,
                )
                r.start()
                sends.append(r)
            for s in range(1, N_DEV):
                src_dev = lax.rem(my - s + N_DEV, N_DEV)
                rows = pl.ds(src_dev * BP, BP)
                recv = pltpu.make_async_remote_copy(
                    src_ref=xfull.at[bi, rows, :],
                    dst_ref=xfull.at[bi, rows, :],
                    send_sem=ag_send.at[bi, s],
                    recv_sem=ag_recv.at[bi, s],
                    device_id=(my,),
                    device_id_type=pl.DeviceIdType.MESH,
                )
                recv.wait_recv()
            for r in sends:
                r.wait_send()

        def reduce_scatter(bi):
            sends = []
            for off in range(1, N_DEV):
                dst = lax.rem(my + off, N_DEV)
                r = pltpu.make_async_remote_copy(
                    src_ref=p_ref.at[dst],
                    dst_ref=rs_buf.at[bi, off],
                    send_sem=rs_send.at[bi, off],
                    recv_sem=rs_recv.at[bi, off],
                    device_id=(dst,),
                    device_id_type=pl.DeviceIdType.MESH,
                )
                r.start()
                sends.append(r)
            rs_buf[bi, 0] = p_ref[my]
            for s in range(1, N_DEV):
                recv = pltpu.make_async_remote_copy(
                    src_ref=p_ref.at[0],
                    dst_ref=rs_buf.at[bi, s],
                    send_sem=rs_send.at[bi, s],
                    recv_sem=rs_recv.at[bi, s],
                    device_id=(my,),
                    device_id_type=pl.DeviceIdType.MESH,
                )
                recv.wait_recv()
            for r in sends:
                r.wait_send()

        xfull[0, my_rows, :] = x_ref[...].astype(jnp.bfloat16)
        all_gather(0)

        layers = [(win0, wout0), (win1, wout1), (win2, wout2)]
        for l, (wi, wo) in enumerate(layers):
            bi = l % 2
            xf = xfull[bi]
            h = jnp.dot(xf, wi[...].astype(jnp.bfloat16),
                        preferred_element_type=jnp.float32)
            h = jnp.maximum(h, 0.0).astype(jnp.bfloat16)
            p = jnp.dot(h, wo[...].astype(jnp.bfloat16),
                        preferred_element_type=jnp.float32)
            p_ref[...] = p.astype(jnp.bfloat16).reshape(N_DEV, BP, D)

            reduce_scatter(bi)
            red = jnp.sum(rs_buf[bi].astype(jnp.float32), axis=0)

            nbi = (l + 1) % 2
            xfull[nbi, my_rows, :] = red.astype(jnp.bfloat16)
            all_gather(nbi)

        out_ref[...] = xfull[1].astype(jnp.float32)

    return pl.pallas_call(
        body,
        out_shape=jax.ShapeDtypeStruct((B, D), jnp.float32),
        in_specs=[pl.BlockSpec(memory_space=pltpu.VMEM)] * 7,
        out_specs=pl.BlockSpec(memory_space=pltpu.VMEM),
        scratch_shapes=[
            pltpu.VMEM((2, B, D), jnp.bfloat16),
            pltpu.VMEM((N_DEV, BP, D), jnp.bfloat16),
            pltpu.VMEM((2, N_DEV, BP, D), jnp.bfloat16),
            pltpu.SemaphoreType.DMA((2, N_DEV)),
            pltpu.SemaphoreType.DMA((2, N_DEV)),
            pltpu.SemaphoreType.DMA((2, N_DEV)),
            pltpu.SemaphoreType.DMA((2, N_DEV)),
        ],
    )(x, Win0, Wout0, Win1, Wout1, Win2, Wout2)


# device time: 64375 ns/iter; 1.1070x vs baseline; 1.1070x over previous
import jax
import jax.numpy as jnp
from jax import lax
from jax.experimental import pallas as pl
from jax.experimental.pallas import tpu as pltpu

N_DEV = 16
BP = 128
B = N_DEV * BP
D = 128
HP = 256


def kernel(x, Win0, Wout0, Win1, Wout1, Win2, Wout2):
    def body(x_ref, win0, wout0, win1, wout1, win2, wout2, out_ref,
             xfull, p_stage, rs_buf, ag_send, ag_recv, rs_send, rs_recv):
        my = lax.axis_index("i")
        my_rows = pl.ds(my * BP, BP)

        def ag_send_all(bi):
            sends = []
            for off in range(1, N_DEV):
                dst = lax.rem(my + off, N_DEV)
                r = pltpu.make_async_remote_copy(
                    src_ref=xfull.at[bi, my_rows, :],
                    dst_ref=xfull.at[bi, my_rows, :],
                    send_sem=ag_send.at[bi, off],
                    recv_sem=ag_recv.at[bi, off],
                    device_id=(dst,),
                    device_id_type=pl.DeviceIdType.MESH,
                )
                r.start()
                sends.append(r)
            return sends

        def ag_wait_one(bi, s):
            src_dev = lax.rem(my - s + N_DEV, N_DEV)
            rows = pl.ds(src_dev * BP, BP)
            pltpu.make_async_remote_copy(
                src_ref=xfull.at[bi, rows, :],
                dst_ref=xfull.at[bi, rows, :],
                send_sem=ag_send.at[bi, s],
                recv_sem=ag_recv.at[bi, s],
                device_id=(my,),
                device_id_type=pl.DeviceIdType.MESH,
            ).wait_recv()
            return rows

        def layer(l, win, wout):
            bi = l % 2
            wi16 = win[...].astype(jnp.bfloat16)
            wo16 = wout[...].astype(jnp.bfloat16)
            rs_sends = []
            for s in range(N_DEV):
                if s == 0:
                    rows = my_rows
                else:
                    rows = ag_wait_one(bi, s)
                xc = xfull[bi, rows, :]
                h = jnp.dot(xc, wi16, preferred_element_type=jnp.float32)
                h = jnp.maximum(h, 0.0).astype(jnp.bfloat16)
                p = jnp.dot(h, wo16, preferred_element_type=jnp.float32)
                p16 = p.astype(jnp.bfloat16)
                if s == 0:
                    rs_buf[bi, 0] = p16
                else:
                    dst = lax.rem(my - s + N_DEV, N_DEV)
                    p_stage[s] = p16
                    r = pltpu.make_async_remote_copy(
                        src_ref=p_stage.at[s],
                        dst_ref=rs_buf.at[bi, s],
                        send_sem=rs_send.at[bi, s],
                        recv_sem=rs_recv.at[bi, s],
                        device_id=(dst,),
                        device_id_type=pl.DeviceIdType.MESH,
                    )
                    r.start()
                    rs_sends.append(r)
            for s in range(1, N_DEV):
                pltpu.make_async_remote_copy(
                    src_ref=p_stage.at[0],
                    dst_ref=rs_buf.at[bi, s],
                    send_sem=rs_send.at[bi, s],
                    recv_sem=rs_recv.at[bi, s],
                    device_id=(my,),
                    device_id_type=pl.DeviceIdType.MESH,
                ).wait_recv()
            red = jnp.sum(rs_buf[bi].astype(jnp.float32), axis=0)

            nbi = (l + 1) % 2
            xfull[nbi, my_rows, :] = red.astype(jnp.bfloat16)
            ag_sends = ag_send_all(nbi)
            return ag_sends, rs_sends

        xfull[0, my_rows, :] = x_ref[...].astype(jnp.bfloat16)
        pending = ag_send_all(0)

        layers = [(win0, wout0), (win1, wout1), (win2, wout2)]
        for l, (wi, wo) in enumerate(layers):
            ag_sends, rs_sends = layer(l, wi, wo)
            for r in pending:
                r.wait_send()
            for r in rs_sends:
                r.wait_send()
            pending = ag_sends

        out_ref[my_rows, :] = xfull[1, my_rows, :].astype(jnp.float32)
        for s in range(1, N_DEV):
            rows = ag_wait_one(1, s)
            out_ref[rows, :] = xfull[1, rows, :].astype(jnp.float32)
        for r in pending:
            r.wait_send()

    return pl.pallas_call(
        body,
        out_shape=jax.ShapeDtypeStruct((B, D), jnp.float32),
        in_specs=[pl.BlockSpec(memory_space=pltpu.VMEM)] * 7,
        out_specs=pl.BlockSpec(memory_space=pltpu.VMEM),
        scratch_shapes=[
            pltpu.VMEM((2, B, D), jnp.bfloat16),
            pltpu.VMEM((N_DEV, BP, D), jnp.bfloat16),
            pltpu.VMEM((2, N_DEV, BP, D), jnp.bfloat16),
            pltpu.SemaphoreType.DMA((2, N_DEV)),
            pltpu.SemaphoreType.DMA((2, N_DEV)),
            pltpu.SemaphoreType.DMA((2, N_DEV)),
            pltpu.SemaphoreType.DMA((2, N_DEV)),
        ],
    )(x, Win0, Wout0, Win1, Wout1, Win2, Wout2)
